# baseline (speedup 1.0000x reference)
import functools

import jax
import jax.numpy as jnp
from jax import lax
from jax.experimental import pallas as pl
from jax.experimental.pallas import tpu as pltpu
from jax.experimental.pallas import tpu_sc as plsc

_N0 = 8192
_N1 = 4096
_TOTAL = _N0 * _N1

_NC = 2
_NS = 16
_NW = _NC * _NS
_L = 16

_PER_W = _TOTAL // _NW
_CHUNK = 32768
_NCHUNK = _PER_W // _CHUNK
_UNROLL = 16
_NITER = _CHUNK // (_L * _UNROLL)

_BIG_I32 = 2**31 - 1

_mesh = plsc.VectorSubcoreMesh(core_axis_name="c", subcore_axis_name="s")


@functools.partial(
    pl.kernel,
    mesh=_mesh,
    out_type=[
        jax.ShapeDtypeStruct((_NW, _L), jnp.float32),
        jax.ShapeDtypeStruct((_NW, _L), jnp.int32),
    ],
    scratch_types=[
        pltpu.VMEM((2, _CHUNK), jnp.float32),
        pltpu.VMEM((_L,), jnp.float32),
        pltpu.VMEM((_L,), jnp.int32),
        pltpu.SemaphoreType.DMA,
        pltpu.SemaphoreType.DMA,
    ],
)
def _sc_partial_argmax(x_hbm, outv_hbm, outi_hbm, buf, stage_v, stage_i,
                       sem0, sem1):
    wid = lax.axis_index("s") * _NC + lax.axis_index("c")
    base = wid * _PER_W
    sems = (sem0, sem1)
    iota = lax.iota(jnp.int32, _L)

    def allmax(x):
        for s in (8, 4, 2, 1):
            x = jnp.maximum(x, jnp.take(x, iota ^ s, mode="promise_in_bounds"))
        return x

    def allmin(x):
        for s in (8, 4, 2, 1):
            x = jnp.minimum(x, jnp.take(x, iota ^ s, mode="promise_in_bounds"))
        return x

    def start(c, slot):
        return pltpu.async_copy(
            x_hbm.at[pl.ds(base + c * _CHUNK, _CHUNK)], buf.at[slot],
            sems[slot])

    pending = [start(0, 0), None]
    m_best = jnp.float32(float("-inf"))
    c_best = jnp.int32(0)
    ninf = jnp.full((_L,), float("-inf"), jnp.float32)

    for c in range(_NCHUNK):
        slot = c & 1
        if c + 1 < _NCHUNK:
            pending[slot ^ 1] = start(c + 1, slot ^ 1)
        pending[slot].wait()
        cbuf = buf.at[slot]

        def mbody(i, accs, cbuf=cbuf):
            a0, a1, a2, a3 = accs
            off = i * (_L * _UNROLL)
            for j in range(_UNROLL):
                v = cbuf[pl.ds(off + j * _L, _L)]
                if j % 4 == 0:
                    a0 = jnp.maximum(a0, v)
                elif j % 4 == 1:
                    a1 = jnp.maximum(a1, v)
                elif j % 4 == 2:
                    a2 = jnp.maximum(a2, v)
                else:
                    a3 = jnp.maximum(a3, v)
            return (a0, a1, a2, a3)

        a0, a1, a2, a3 = lax.fori_loop(0, _NITER, mbody,
                                       (ninf, ninf, ninf, ninf))
        m = allmax(jnp.maximum(jnp.maximum(a0, a1), jnp.maximum(a2, a3)))[0]
        better = m > m_best
        c_best = jnp.where(better, jnp.int32(c), c_best)
        m_best = jnp.where(better, m, m_best)

    pltpu.sync_copy(x_hbm.at[pl.ds(base + c_best * _CHUNK, _CHUNK)],
                    buf.at[0])
    cbuf0 = buf.at[0]
    mvec = jnp.broadcast_to(m_best, (_L,))
    big = jnp.full((_L,), _BIG_I32, jnp.int32)

    def rbody(i, pmins):
        p0, p1 = pmins
        off = i * (_L * _UNROLL)
        for j in range(_UNROLL):
            v = cbuf0[pl.ds(off + j * _L, _L)]
            pos = iota + (off + j * _L)
            cand = jnp.where(v == mvec, pos, _BIG_I32)
            if j % 2 == 0:
                p0 = jnp.minimum(p0, cand)
            else:
                p1 = jnp.minimum(p1, cand)
        return (p0, p1)

    p0, p1 = lax.fori_loop(0, _NITER, rbody, (big, big))
    pos = allmin(jnp.minimum(p0, p1))[0]
    flat = base + c_best * _CHUNK + pos

    stage_v[...] = jnp.broadcast_to(m_best, (_L,))
    stage_i[...] = jnp.broadcast_to(flat, (_L,))
    pltpu.sync_copy(stage_v, outv_hbm.at[wid])
    pltpu.sync_copy(stage_i, outi_hbm.at[wid])


def _combine_body(v_ref, i_ref, o_ref):
    vals = v_ref[...]
    idxs = i_ref[...]
    m = jnp.max(vals)
    cand = jnp.where(vals == m, idxs, _BIG_I32)
    idx = jnp.min(cand)
    o_ref[0] = idx // _N0
    o_ref[1] = idx % _N1


_combine = pl.pallas_call(
    _combine_body,
    out_shape=jax.ShapeDtypeStruct((2,), jnp.int32),
    in_specs=[
        pl.BlockSpec(memory_space=pltpu.VMEM),
        pl.BlockSpec(memory_space=pltpu.VMEM),
    ],
    out_specs=pl.BlockSpec(memory_space=pltpu.SMEM),
)


def kernel(inputs):
    flat = jnp.reshape(inputs, (_TOTAL,))
    vals, idxs = _sc_partial_argmax(flat)
    return _combine(vals, idxs)


# speedup vs baseline: 2.4932x; 2.4932x over previous
import functools

import jax
import jax.numpy as jnp
from jax import lax
from jax.experimental import pallas as pl
from jax.experimental.pallas import tpu as pltpu
from jax.experimental.pallas import tpu_sc as plsc

_N0 = 8192
_N1 = 4096

_NC = 2
_NS = 16
_NW = _NC * _NS
_L = 16

_ROWS_W = _N0 // _NW
_CROWS = 8
_NCHUNK = _ROWS_W // _CROWS
_UNROLL = 16
_NITER = _N1 // (_L * _UNROLL)

_BIG_I32 = 2**31 - 1

_mesh = plsc.VectorSubcoreMesh(core_axis_name="c", subcore_axis_name="s")


@functools.partial(
    pl.kernel,
    mesh=_mesh,
    out_type=[
        jax.ShapeDtypeStruct((_NW, _L), jnp.float32),
        jax.ShapeDtypeStruct((_NW, _L), jnp.int32),
    ],
    scratch_types=[
        pltpu.VMEM((2, _CROWS, _N1), jnp.float32),
        pltpu.VMEM((_L,), jnp.float32),
        pltpu.VMEM((_L,), jnp.int32),
        pltpu.SemaphoreType.DMA,
        pltpu.SemaphoreType.DMA,
    ],
    compiler_params=pltpu.CompilerParams(use_tc_tiling_on_sc=True),
)
def _sc_partial_argmax(x_hbm, outv_hbm, outi_hbm, buf, stage_v, stage_i,
                       sem0, sem1):
    wid = lax.axis_index("s") * _NC + lax.axis_index("c")
    row_base = wid * _ROWS_W
    sems = (sem0, sem1)
    iota = lax.iota(jnp.int32, _L)

    def allmax(x):
        for s in (8, 4, 2, 1):
            x = jnp.maximum(x, x.at[iota ^ s].get(mode="promise_in_bounds"))
        return x

    def allmin(x):
        for s in (8, 4, 2, 1):
            x = jnp.minimum(x, x.at[iota ^ s].get(mode="promise_in_bounds"))
        return x

    def start(c, slot):
        return pltpu.async_copy(
            x_hbm.at[pl.ds(row_base + c * _CROWS, _CROWS)], buf.at[slot],
            sems[slot])

    def wait_for(c, slot):
        pltpu.make_async_copy(
            x_hbm.at[pl.ds(row_base + c * _CROWS, _CROWS)], buf.at[slot],
            sems[slot]).wait()

    ninf = jnp.full((_L,), float("-inf"), jnp.float32)

    start(0, 0)
    start(1, 1)

    def chunk_max(slot):
        acc = (ninf, ninf, ninf, ninf)
        for r in range(_CROWS):
            rbuf = buf.at[slot, r]

            def mbody(i, accs, rbuf=rbuf):
                a0, a1, a2, a3 = accs
                off = i * (_L * _UNROLL)
                for j in range(_UNROLL):
                    v = rbuf[pl.ds(off + j * _L, _L)]
                    if j % 4 == 0:
                        a0 = jnp.maximum(a0, v)
                    elif j % 4 == 1:
                        a1 = jnp.maximum(a1, v)
                    elif j % 4 == 2:
                        a2 = jnp.maximum(a2, v)
                    else:
                        a3 = jnp.maximum(a3, v)
                return (a0, a1, a2, a3)

            acc = lax.fori_loop(0, _NITER, mbody, acc)
        a0, a1, a2, a3 = acc
        return allmax(jnp.maximum(jnp.maximum(a0, a1),
                                  jnp.maximum(a2, a3)))[0]

    def cbody(k, carry):
        m_best, c_best = carry
        for slot in (0, 1):
            c = 2 * k + slot
            wait_for(c, slot)
            m = chunk_max(slot)
            nxt = c + 2

            @pl.when(nxt < _NCHUNK)
            def _():
                start(nxt, slot)

            better = m > m_best
            c_best = jnp.where(better, c, c_best)
            m_best = jnp.where(better, m, m_best)
        return (m_best, c_best)

    m_best, c_best = lax.fori_loop(
        0, _NCHUNK // 2, cbody,
        (jnp.float32(float("-inf")), jnp.int32(0)))

    win_row = row_base + c_best * _CROWS
    pltpu.sync_copy(x_hbm.at[pl.ds(win_row, _CROWS)], buf.at[0])
    mvec = jnp.broadcast_to(m_best, (_L,))
    big = jnp.full((_L,), _BIG_I32, jnp.int32)

    pmin = (big, big)
    for r in range(_CROWS):
        rbuf = buf.at[0, r]
        flat_row = (win_row + r) * _N1

        def rbody(i, pmins, rbuf=rbuf, flat_row=flat_row):
            p0, p1 = pmins
            off = i * (_L * _UNROLL)
            for j in range(_UNROLL):
                v = rbuf[pl.ds(off + j * _L, _L)]
                pos = iota + (flat_row + off + j * _L)
                cand = jnp.where(v == mvec, pos, _BIG_I32)
                if j % 2 == 0:
                    p0 = jnp.minimum(p0, cand)
                else:
                    p1 = jnp.minimum(p1, cand)
            return (p0, p1)

        pmin = lax.fori_loop(0, _NITER, rbody, pmin)

    flat = allmin(jnp.minimum(pmin[0], pmin[1]))[0]

    stage_v[...] = jnp.broadcast_to(m_best, (_L,))
    stage_i[...] = jnp.broadcast_to(flat, (_L,))
    pltpu.sync_copy(stage_v, outv_hbm.at[wid])
    pltpu.sync_copy(stage_i, outi_hbm.at[wid])


def _combine_body(v_ref, i_ref, o_ref):
    vals = v_ref[...]
    idxs = i_ref[...]
    m = jnp.max(vals)
    cand = jnp.where(vals == m, idxs, _BIG_I32)
    idx = jnp.min(cand)
    o_ref[0] = idx // _N0
    o_ref[1] = idx % _N1


_combine = pl.pallas_call(
    _combine_body,
    out_shape=jax.ShapeDtypeStruct((2,), jnp.int32),
    in_specs=[
        pl.BlockSpec(memory_space=pltpu.VMEM),
        pl.BlockSpec(memory_space=pltpu.VMEM),
    ],
    out_specs=pl.BlockSpec(memory_space=pltpu.SMEM),
)


def kernel(inputs):
    vals, idxs = _sc_partial_argmax(inputs)
    return _combine(vals, idxs)
